# baseline (speedup 1.0000x reference)
import functools

import jax
import jax.numpy as jnp
from jax import lax
from jax.experimental import pallas as pl
from jax.experimental.pallas import tpu as pltpu
from jax.experimental.pallas import tpu_sc as plsc

N = 10000
HN = N // 2
D = 256
NUM_CORES = 2
NUM_SUBCORES = 16
NW = NUM_CORES * NUM_SUBCORES
CHUNK = 32
NBUF = 2
HN_PAD = 5120
ROWS_PER_TILE = HN_PAD // NUM_SUBCORES
LCAP = 5152


def _matmul_body(x_ref, w_ref, o_ref):
    o_ref[...] = lax.dot_general(
        x_ref[...], w_ref[...], (((1,), (1,)), ((), ())),
        preferred_element_type=jnp.float32)


def _support(x, w):
    n = x.shape[0]
    bn = 1000
    nb = n // bn
    return pl.pallas_call(
        _matmul_body,
        grid=(nb,),
        in_specs=[
            pl.BlockSpec((bn, D), lambda i: (i, 0)),
            pl.BlockSpec((D, D), lambda i: (0, 0)),
        ],
        out_specs=pl.BlockSpec((bn, D), lambda i: (i, 0)),
        out_shape=jax.ShapeDtypeStruct((n, D), jnp.float32),
    )(x, w)


def _sc_aggregate(sup, src, dst, val):
    epw = src.shape[0] // NW
    ngrp = epw // 16
    tail = epw - ngrp * 16
    mesh = plsc.VectorSubcoreMesh(core_axis_name="c", subcore_axis_name="s")

    @functools.partial(
        pl.kernel,
        mesh=mesh,
        out_type=jax.ShapeDtypeStruct((2, NUM_CORES, HN_PAD, D), jnp.float32),
        scratch_types=[
            pltpu.VMEM((LCAP,), jnp.int32),
            pltpu.VMEM((LCAP,), jnp.int32),
            pltpu.VMEM((LCAP,), jnp.float32),
            pltpu.VMEM((CHUNK, D), jnp.float32),
            pltpu.VMEM((CHUNK, D), jnp.float32),
            pltpu.VMEM((CHUNK, 128), jnp.float32),
            pltpu.VMEM((CHUNK, 128), jnp.float32),
            pltpu.VMEM((CHUNK, 128), jnp.float32),
            pltpu.VMEM((CHUNK, 128), jnp.float32),
            pltpu.VMEM_SHARED((HN_PAD, 128), jnp.float32),
            pltpu.VMEM_SHARED((HN_PAD, 128), jnp.float32),
            pltpu.SemaphoreType.DMA,
            pltpu.SemaphoreType.DMA,
            pltpu.SemaphoreType.DMA,
            pltpu.SemaphoreType.DMA,
        ],
    )
    def k(sup_hbm, src_hbm, dst_hbm, val_hbm, out_hbm,
          lsrc, ldst, lval, grow0, grow1, sa0, sa1, sb0, sb1, accA, accB,
          g0, g1, s0, s1):
        grows = (grow0, grow1)
        sas = (sa0, sa1)
        sbs = (sb0, sb1)
        gsem = (g0, g1)
        ssem = (s0, s1)
        c = lax.axis_index("c")
        s = lax.axis_index("s")
        w = c * NUM_SUBCORES + s
        base = s * ROWS_PER_TILE
        elo = w * epw

        def stage_raw():
            pltpu.sync_copy(src_hbm.at[pl.ds(elo, epw)],
                            lsrc.at[pl.ds(0, epw)])
            pltpu.sync_copy(dst_hbm.at[pl.ds(elo, epw)],
                            ldst.at[pl.ds(0, epw)])
            pltpu.sync_copy(val_hbm.at[pl.ds(elo, epw)],
                            lval.at[pl.ds(0, epw)])

        lanes = lax.iota(jnp.int32, 16)
        hn16 = jnp.broadcast_to(HN, (16,)).astype(jnp.int32)
        one16 = jnp.broadcast_to(1, (16,)).astype(jnp.int32)
        zi = jnp.zeros((16,), jnp.int32)
        zf = jnp.zeros((16,), jnp.float32)

        def take16(v, idx):
            return lax.gather(
                v, idx[:, None],
                lax.GatherDimensionNumbers(
                    offset_dims=(), collapsed_slice_dims=(0,),
                    start_index_map=(0,)),
                slice_sizes=(1,),
                mode=lax.GatherScatterMode.PROMISE_IN_BOUNDS)

        def bit(x, b):
            return lax.bitwise_and(lax.shift_right_logical(x, b), one16)

        def extract(hi_pass):
            def handle_group(s16, d16, v16, ptr):
                lt0 = lax.shift_right_logical(d16 - hn16, 31)
                want = one16 - lt0 if hi_pass else lt0
                cnt = want
                for sh in (8, 4, 2, 1):
                    prm = lax.bitwise_xor(
                        lanes, jnp.broadcast_to(sh, (16,)).astype(jnp.int32))
                    cnt = cnt + take16(cnt, prm)
                n_want = cnt[0]
                key = (one16 - want) * jnp.broadcast_to(
                    16, (16,)).astype(jnp.int32) + lanes
                for kk, lg2k in ((1, 0), (2, 1), (4, 2), (8, 3)):
                    down = bit(lanes, lg2k + 1)
                    j = kk
                    lgj = lg2k
                    while j >= 1:
                        j16 = jnp.broadcast_to(j, (16,)).astype(jnp.int32)
                        t = take16(key, lax.bitwise_xor(lanes, j16))
                        mn = jnp.minimum(key, t)
                        mx = jnp.maximum(key, t)
                        sel = lax.bitwise_xor(bit(lanes, lgj), down)
                        key = mn * (one16 - sel) + mx * sel
                        j >>= 1
                        lgj -= 1
                perm = lax.bitwise_and(
                    key, jnp.broadcast_to(15, (16,)).astype(jnp.int32))
                dsh = d16 - hn16 if hi_pass else d16
                lsrc[pl.ds(ptr, 16)] = take16(s16, perm)
                ldst[pl.ds(ptr, 16)] = take16(dsh, perm)
                lval[pl.ds(ptr, 16)] = take16(v16, perm)
                return ptr + n_want

            def part_body(g, ptr):
                g0idx = g * 16
                return handle_group(lsrc[pl.ds(g0idx, 16)],
                                    ldst[pl.ds(g0idx, 16)],
                                    lval[pl.ds(g0idx, 16)], ptr)

            ptr = lax.fori_loop(0, ngrp, part_body, jnp.int32(0))
            if tail:
                g0idx = epw - 16
                keep_i = one16 - lax.shift_right_logical(
                    lanes - jnp.broadcast_to(16 - tail, (16,)).astype(
                        jnp.int32), 31)
                keep_f = keep_i.astype(jnp.float32)
                ptr = handle_group(lsrc[pl.ds(g0idx, 16)] * keep_i,
                                   ldst[pl.ds(g0idx, 16)] * keep_i,
                                   lval[pl.ds(g0idx, 16)] * keep_f, ptr)
            return ptr

        def zero_tail(n_p):
            b0 = (n_p >> 4) << 4
            keep_i = lax.shift_right_logical(
                lanes - jnp.broadcast_to(n_p - b0, (16,)).astype(jnp.int32),
                31)
            keep_f = keep_i.astype(jnp.float32)
            sl = pl.ds(b0, 16)
            lsrc[sl] = lsrc[sl] * keep_i
            ldst[sl] = ldst[sl] * keep_i
            lval[sl] = lval[sl] * keep_f
            for t in range(1, 10):
                slt = pl.ds(b0 + t * 16, 16)
                lsrc[slt] = zi
                ldst[slt] = zi
                lval[slt] = zf

        def zero_body(i, carry):
            for r in range(128 // 16):
                sa0[i, pl.ds(r * 16, 16)] = zf
            return carry

        def run_pass(n_p, p):
            nc_p = (n_p + CHUNK - 1) >> 5
            ncu = (nc_p >> 1) * 2 + 1

            lax.fori_loop(0, CHUNK, zero_body, 0)
            for t in range(ROWS_PER_TILE // CHUNK):
                sl = pl.ds(base + t * CHUNK, CHUNK)
                pltpu.sync_copy(sa0, accA.at[sl])
                pltpu.sync_copy(sa0, accB.at[sl])
            plsc.subcore_barrier()

            def gather(j, b):
                idx = lsrc.at[pl.ds(j * CHUNK, CHUNK)]
                return pltpu.make_async_copy(sup_hbm.at[idx], grows[b],
                                             gsem[b])

            def scatter_h(j, b, sbuf, accX):
                idx = ldst.at[pl.ds(j * CHUNK, CHUNK)]
                return pltpu.make_async_copy(sbuf[b], accX.at[idx], ssem[b])

            def scatter_start(j, b):
                scatter_h(j, b, sas, accA).start(add=True)
                scatter_h(j, b, sbs, accB).start(add=True)

            def scatter_wait(j, b):
                scatter_h(j, b, sas, accA).wait()
                scatter_h(j, b, sbs, accB).wait()

            def scale(j, b):
                for g in range(CHUNK // 16):
                    vals16 = lval[pl.ds(j * CHUNK + g * 16, 16)]
                    for e16 in range(16):
                        e = g * 16 + e16
                        v16 = jnp.broadcast_to(vals16[e16], (16,))
                        for r in range(D // 16):
                            sl = pl.ds((r % 8) * 16, 16)
                            dstbuf = sas[b] if r < 8 else sbs[b]
                            dstbuf[e, sl] = (
                                grows[b][e, pl.ds(r * 16, 16)] * v16)

            gather(0, 0).start()
            gather(1, 1).start()
            gather(0, 0).wait()
            scale(jnp.int32(0), 0)
            scatter_start(0, 0)

            def block_body(t, carry):
                j0 = 1 + t * NBUF
                for u in range(NBUF):
                    j = j0 + u
                    b = (1 + u) % NBUF
                    bprev = u % NBUF
                    scatter_wait(j - 1, bprev)
                    gather(j + 1, bprev).start()
                    gather(j, b).wait()
                    scale(j, b)
                    scatter_start(j, b)
                return carry

            lax.fori_loop(0, (ncu - 1) // NBUF, block_body, 0)
            scatter_wait(ncu - 1, 0)
            gather(ncu, 1).wait()

            plsc.subcore_barrier()
            pltpu.sync_copy(accA.at[pl.ds(base, ROWS_PER_TILE)],
                            out_hbm.at[p, c, pl.ds(base, ROWS_PER_TILE),
                                       pl.ds(0, 128)])
            pltpu.sync_copy(accB.at[pl.ds(base, ROWS_PER_TILE)],
                            out_hbm.at[p, c, pl.ds(base, ROWS_PER_TILE),
                                       pl.ds(128, 128)])
            plsc.subcore_barrier()

        stage_raw()
        n_lo = extract(False)
        zero_tail(n_lo)
        run_pass(n_lo, 0)
        stage_raw()
        n_hi = extract(True)
        zero_tail(n_hi)
        run_pass(n_hi, 1)

    return k(sup, src, dst, val)


@jax.jit
def kernel(x, adj_indices, adj_values, W):
    sup = _support(x, W)
    out = _sc_aggregate(sup, adj_indices[1], adj_indices[0], adj_values)
    lo = out[0, 0, :HN] + out[0, 1, :HN]
    hi = out[1, 0, :HN] + out[1, 1, :HN]
    return jnp.concatenate([lo, hi], axis=0)


# speedup vs baseline: 1.4697x; 1.4697x over previous
import functools

import jax
import jax.numpy as jnp
from jax import lax
from jax.experimental import pallas as pl
from jax.experimental.pallas import tpu as pltpu
from jax.experimental.pallas import tpu_sc as plsc

N = 10000
D = 256
HALF = 128
NUM_CORES = 2
NUM_SUBCORES = 16
CHUNK = 40
NBUF = 3
N_PAD = 10240
ROWS_PER_TILE = N_PAD // NUM_SUBCORES


def _matmul_body(x_ref, w_ref, o_ref):
    o_ref[...] = lax.dot_general(
        x_ref[...], w_ref[...], (((1,), (1,)), ((), ())),
        preferred_element_type=jnp.float32)


def _support_stacked(x, w):
    n = x.shape[0]
    bn = 1000
    nb = n // bn
    return pl.pallas_call(
        _matmul_body,
        grid=(NUM_CORES, nb),
        in_specs=[
            pl.BlockSpec((bn, D), lambda h, i: (i, 0)),
            pl.BlockSpec((HALF, D), lambda h, i: (h, 0)),
        ],
        out_specs=pl.BlockSpec((bn, HALF), lambda h, i, _nb=nb: (h * _nb + i, 0)),
        out_shape=jax.ShapeDtypeStruct((NUM_CORES * n, HALF), jnp.float32),
    )(x, w)


def _sc_aggregate(sup, src, dst, val):
    e_per_tile = src.shape[0] // NUM_SUBCORES
    nc = e_per_tile // CHUNK
    assert nc * CHUNK == e_per_tile and (nc - 1) % NBUF == 0
    mesh = plsc.VectorSubcoreMesh(core_axis_name="c", subcore_axis_name="s")

    @functools.partial(
        pl.kernel,
        mesh=mesh,
        out_type=jax.ShapeDtypeStruct((N, D), jnp.float32),
        scratch_types=[
            pltpu.VMEM(((nc + 2) * CHUNK,), jnp.int32),
            pltpu.VMEM((nc * CHUNK,), jnp.int32),
            pltpu.VMEM((nc * CHUNK,), jnp.float32),
            pltpu.VMEM((CHUNK, HALF), jnp.float32),
            pltpu.VMEM((CHUNK, HALF), jnp.float32),
            pltpu.VMEM((CHUNK, HALF), jnp.float32),
            pltpu.VMEM_SHARED((N_PAD, HALF), jnp.float32),
            pltpu.SemaphoreType.DMA,
            pltpu.SemaphoreType.DMA,
            pltpu.SemaphoreType.DMA,
            pltpu.SemaphoreType.DMA,
            pltpu.SemaphoreType.DMA,
            pltpu.SemaphoreType.DMA,
            pltpu.SemaphoreType.DMA,
            pltpu.SemaphoreType.DMA,
            pltpu.SemaphoreType.DMA,
        ],
    )
    def k(sup_hbm, src_hbm, dst_hbm, val_hbm, out_hbm,
          src_v, dst_v, val_v, rows0, rows1, rows2, acc,
          g0, g1, g2, s0, s1, s2, h0, h1, h2):
        rows = (rows0, rows1, rows2)
        gsem = (g0, g1, g2)
        ssem = (s0, s1, s2)
        gsem2 = (h0, h1, h2)
        c = lax.axis_index("c")
        s = lax.axis_index("s")
        base = s * ROWS_PER_TILE
        elo = s * e_per_tile
        pltpu.sync_copy(src_hbm.at[pl.ds(elo, e_per_tile)],
                        src_v.at[pl.ds(0, e_per_tile)])
        pltpu.sync_copy(dst_hbm.at[pl.ds(elo, e_per_tile)], dst_v)
        pltpu.sync_copy(val_hbm.at[pl.ds(elo, e_per_tile)], val_v)

        coff = jnp.broadcast_to(c * N, (16,)).astype(jnp.int32)

        def off_body(i, carry):
            sl = pl.ds(i * 16, 16)
            src_v[sl] = src_v[sl] + coff
            return carry

        lax.fori_loop(0, e_per_tile // 16, off_body, 0)
        for t in range(2 * CHUNK // 16):
            src_v[pl.ds(e_per_tile + t * 16, 16)] = jnp.zeros((16,), jnp.int32)

        def zero_body(i, carry):
            for r in range(HALF // 16):
                rows0[i, pl.ds(r * 16, 16)] = jnp.zeros((16,), jnp.float32)
            return carry

        lax.fori_loop(0, CHUNK, zero_body, 0)
        for t in range(ROWS_PER_TILE // CHUNK):
            pltpu.sync_copy(rows0, acc.at[pl.ds(base + t * CHUNK, CHUNK)])
        plsc.subcore_barrier()

        def gather_a(j, b):
            idx = src_v.at[pl.ds(j * CHUNK, 24)]
            return pltpu.make_async_copy(
                sup_hbm.at[idx], rows[b].at[pl.ds(0, 24)], gsem[b])

        def gather_b(j, b):
            idx = src_v.at[pl.ds(j * CHUNK + 24, 16)]
            return pltpu.make_async_copy(
                sup_hbm.at[idx], rows[b].at[pl.ds(24, 16)], gsem2[b])

        class _G:
            def __init__(self, j, b):
                self.j, self.b = j, b

            def start(self):
                gather_a(self.j, self.b).start()
                gather_b(self.j, self.b).start()

            def wait(self):
                gather_a(self.j, self.b).wait()
                gather_b(self.j, self.b).wait()

        def gather(j, b):
            return _G(j, b)

        def scatter(j, b):
            idx = dst_v.at[pl.ds(j * CHUNK, CHUNK)]
            return pltpu.make_async_copy(rows[b], acc.at[idx], ssem[b])

        def scale(j, b):
            def group_body(g, carry):
                e0 = g * 16
                vals16 = val_v[pl.ds(j * CHUNK + e0, 16)]
                for e16 in range(16):
                    v16 = jnp.broadcast_to(vals16[e16], (16,))
                    for r in range(HALF // 16):
                        sl = pl.ds(r * 16, 16)
                        rows[b][e0 + e16, sl] = rows[b][e0 + e16, sl] * v16
                return carry

            lax.fori_loop(0, CHUNK // 16, group_body, 0)
            tail = CHUNK % 16
            if tail:
                t0 = CHUNK - 16
                vals16 = val_v[pl.ds(j * CHUNK + t0, 16)]
                for e16 in range(16 - tail, 16):
                    v16 = jnp.broadcast_to(vals16[e16], (16,))
                    for r in range(HALF // 16):
                        sl = pl.ds(r * 16, 16)
                        rows[b][t0 + e16, sl] = rows[b][t0 + e16, sl] * v16

        gather(0, 0).start()
        gather(1, 1).start()
        gather(2, 2).start()
        gather(0, 0).wait()
        scale(jnp.int32(0), 0)
        scatter(0, 0).start(add=True)

        def block_body(t, carry):
            j0 = 1 + t * NBUF
            for u in range(NBUF):
                j = j0 + u
                b = (1 + u) % NBUF
                bprev = u % NBUF
                scatter(j - 1, bprev).wait()
                gather(j + 2, bprev).start()
                gather(j, b).wait()
                scale(j, b)
                scatter(j, b).start(add=True)
            return carry

        lax.fori_loop(0, (nc - 1) // NBUF, block_body, 0)
        scatter(nc - 1, (nc - 1) % NBUF).wait()
        gather(nc, nc % NBUF).wait()
        gather(nc + 1, (nc + 1) % NBUF).wait()

        plsc.subcore_barrier()
        col = pl.multiple_of(c * HALF, HALF)
        last = NUM_SUBCORES - 1

        @pl.when(s < last)
        def _():
            pltpu.sync_copy(acc.at[pl.ds(base, ROWS_PER_TILE)],
                            out_hbm.at[pl.ds(base, ROWS_PER_TILE),
                                       pl.ds(col, HALF)])

        @pl.when(s == last)
        def _():
            tail = N - last * ROWS_PER_TILE
            pltpu.sync_copy(acc.at[pl.ds(base, tail)],
                            out_hbm.at[pl.ds(base, tail), pl.ds(col, HALF)])

    return k(sup, src, dst, val)


@jax.jit
def kernel(x, adj_indices, adj_values, W):
    sup = _support_stacked(x, W)
    return _sc_aggregate(sup, adj_indices[1], adj_indices[0], adj_values)
